# baseline (speedup 1.0000x reference)
import functools

import jax
import jax.numpy as jnp
from jax import lax
from jax.experimental import pallas as pl
from jax.experimental.pallas import tpu as pltpu
from jax.experimental.pallas import tpu_sc as plsc

_L = 16


def _sc_embed(idx_hbm, table_hbm, scales_hbm, out_hbm,
              idx_v, sidx_v, buf_v, out_v, sc_v, sem,
              *, b_per_w, d):
    n_chunks = b_per_w * d // _L
    pts_per_chunk = _L // d

    wid = lax.axis_index("s") * 2 + lax.axis_index("c")
    base = wid * b_per_w

    pltpu.sync_copy(scales_hbm, sc_v)
    pltpu.sync_copy(idx_hbm.at[pl.ds(base, b_per_w)], idx_v)

    def sbody(i, carry):
        ids = idx_v[pl.ds(i * _L, _L)]
        sidx_v[pl.ds(i * _L, _L)] = lax.shift_right_logical(ids, 4)
        return carry

    lax.fori_loop(0, b_per_w // _L, sbody, 0)

    pltpu.async_copy(table_hbm.at[sidx_v], buf_v, sem).wait()

    s = sc_v[...]
    lane = lax.iota(jnp.int32, _L)
    lane_pt = lax.shift_right_logical(lane, 3)
    lane_col = lax.bitwise_and(lane, d - 1)

    def body(g, carry):
        pvec = lane_pt + g * pts_per_chunk
        ids = plsc.load_gather(idx_v, [pvec])
        col = lax.bitwise_and(ids, 15) * d + lane_col
        v = plsc.load_gather(buf_v, [pvec, col])
        out_v[pl.ds(g * _L, _L)] = v * s
        return carry

    lax.fori_loop(0, n_chunks, body, 0)
    pltpu.sync_copy(out_v, out_hbm.at[pl.ds(base * d, b_per_w * d)])


def kernel(point_id, cffs_scaled, cff_scales):
    b = point_id.shape[0]
    v, d = cffs_scaled.shape
    nw = 32
    b_per_w = b // nw
    rows_per_super = 128 // d

    idx = point_id.astype(jnp.int32)
    table128 = cffs_scaled.reshape(v // rows_per_super, 128)
    scales16 = jnp.tile(cff_scales, _L // d)

    run = pl.kernel(
        functools.partial(_sc_embed, b_per_w=b_per_w, d=d),
        out_type=jax.ShapeDtypeStruct((b * d,), jnp.float32),
        mesh=plsc.VectorSubcoreMesh(core_axis_name="c", subcore_axis_name="s"),
        compiler_params=pltpu.CompilerParams(needs_layout_passes=False),
        scratch_types=[
            pltpu.VMEM((b_per_w,), jnp.int32),
            pltpu.VMEM((b_per_w,), jnp.int32),
            pltpu.VMEM((b_per_w, 128), jnp.float32),
            pltpu.VMEM((b_per_w * d,), jnp.float32),
            pltpu.VMEM((_L,), jnp.float32),
            pltpu.SemaphoreType.DMA,
        ],
    )
    out = run(idx, table128, scales16)
    return out.reshape(b, d)


# speedup vs baseline: 7.4209x; 7.4209x over previous
import functools

import jax
import jax.numpy as jnp
from jax import lax
from jax.experimental import pallas as pl
from jax.experimental.pallas import tpu as pltpu
from jax.experimental.pallas import tpu_sc as plsc

_L = 16
_PP = 32
_TC = 128


def _sc_embed(idx_hbm, tablet_hbm, scales_hbm, out_hbm,
              idx_v, buf_a, buf_b, out_v, sc_v, sem_a, sem_b,
              *, b_per_w, d, v_cols):
    n_phases = b_per_w // _PP
    max_tile_start = ((v_cols - 1) // _TC) * _TC

    wid = lax.axis_index("s") * 2 + lax.axis_index("c")
    base = wid * b_per_w

    pltpu.sync_copy(scales_hbm, sc_v)
    pltpu.sync_copy(idx_hbm.at[pl.ds(base, b_per_w)], idx_v)

    lane = lax.iota(jnp.int32, _L)
    lane_pt = lax.shift_right_logical(lane, 3)
    lane_col = lax.bitwise_and(lane, d - 1)
    s = sc_v[...]

    def issue_phase(p, buf, sem):
        for h in range(_PP // _L):
            ids = idx_v[pl.ds(p * _PP + h * _L, _L)]
            starts = lax.shift_left(lax.shift_right_logical(ids, 7), 7)
            for l in range(_L):
                c0 = jnp.sum(jnp.where(lane == l, starts, 0))
                c0 = lax.min(lax.max(c0, 0), max_tile_start)
                c0 = pl.multiple_of(c0, _TC)
                pltpu.async_copy(
                    tablet_hbm.at[:, pl.ds(c0, _TC)],
                    buf.at[:, pl.ds((h * _L + l) * _TC, _TC)],
                    sem,
                )

    def drain_phase(buf, sem):
        pltpu.make_async_copy(
            tablet_hbm.at[:, pl.ds(0, _PP * _TC)], buf, sem
        ).wait()

    def extract_phase(p, buf):
        def body(g, carry):
            slot = lane_pt + g * 2
            ids = plsc.load_gather(idx_v, [p * _PP + slot])
            col = slot * _TC + lax.bitwise_and(ids, _TC - 1)
            v = plsc.load_gather(buf, [lane_col, col])
            out_v[pl.ds(p * _PP * d + g * _L, _L)] = v * s
            return carry

        lax.fori_loop(0, _PP * d // _L, body, 0)

    issue_phase(0, buf_a, sem_a)

    def phase_loop(p, carry):
        even = lax.rem(p, 2) == 0

        @pl.when(even)
        def _():
            @pl.when(p + 1 < n_phases)
            def _():
                issue_phase_dyn(p + 1, buf_b, sem_b)
            drain_phase(buf_a, sem_a)
            extract_phase_dyn(p, buf_a)

        @pl.when(jnp.logical_not(even))
        def _():
            @pl.when(p + 1 < n_phases)
            def _():
                issue_phase_dyn(p + 1, buf_a, sem_a)
            drain_phase(buf_b, sem_b)
            extract_phase_dyn(p, buf_b)

        return carry

    def issue_phase_dyn(p, buf, sem):
        issue_phase(p, buf, sem)

    def extract_phase_dyn(p, buf):
        extract_phase(p, buf)

    lax.fori_loop(0, n_phases, phase_loop, 0)
    pltpu.sync_copy(out_v, out_hbm.at[pl.ds(base * d, b_per_w * d)])


def kernel(point_id, cffs_scaled, cff_scales):
    b = point_id.shape[0]
    v, d = cffs_scaled.shape
    nw = 32
    b_per_w = b // nw

    idx = point_id.astype(jnp.int32)
    tablet = cffs_scaled.T
    scales16 = jnp.tile(cff_scales, _L // d)

    run = pl.kernel(
        functools.partial(_sc_embed, b_per_w=b_per_w, d=d, v_cols=v),
        out_type=jax.ShapeDtypeStruct((b * d,), jnp.float32),
        mesh=plsc.VectorSubcoreMesh(core_axis_name="c", subcore_axis_name="s"),
        compiler_params=pltpu.CompilerParams(needs_layout_passes=False),
        scratch_types=[
            pltpu.VMEM((b_per_w,), jnp.int32),
            pltpu.VMEM((d, _PP * _TC), jnp.float32),
            pltpu.VMEM((d, _PP * _TC), jnp.float32),
            pltpu.VMEM((b_per_w * d,), jnp.float32),
            pltpu.VMEM((_L,), jnp.float32),
            pltpu.SemaphoreType.DMA,
            pltpu.SemaphoreType.DMA,
        ],
    )
    out = run(idx, tablet, scales16)
    return out.reshape(b, d)
